# baseline (speedup 1.0000x reference)
import functools

import jax
import jax.numpy as jnp
from jax import lax
from jax.experimental import pallas as pl
from jax.experimental.pallas import tpu as pltpu
from jax.experimental.pallas import tpu_sc as plsc

N = 10000
E = 320000
D = 128
H1 = 32
H2 = 16
C = 7

NC = 2
NS = 16
NW = NC * NS
K = 128
NCHUNKS = E // K
CA = 80
CB = 76
SLAB = CA + 4
NROW = N // NS
CP = 8


def _sc_mesh():
    return plsc.VectorSubcoreMesh(
        core_axis_name="c", subcore_axis_name="s",
        num_cores=NC, num_subcores=NS,
    )


def _sc_params(n_in):
    return pltpu.CompilerParams(
        needs_layout_passes=False,
        use_tc_tiling_on_sc=False,
        allow_input_fusion=[True] * n_in,
    )


_SC_PARAMS = _sc_params(4)


def _agg_split(c, s):
    is0 = c == 0
    count = jnp.where(is0, CA + 4 * (s == 0).astype(jnp.int32), CB)
    start = jnp.where(
        is0,
        jnp.where(s == 0, 0, 4 + s * CA),
        16 * CA + 4 + s * CB,
    )
    dma_start = jnp.minimum(start, NCHUNKS - SLAB)
    return count, dma_start, start - dma_start


def _deg_split(wid):
    count = 78 + (wid < 4).astype(jnp.int32)
    start = wid * 78 + jnp.minimum(wid, 4)
    dma_start = jnp.minimum(start, NCHUNKS - 79)
    return count, dma_start, start - dma_start


@functools.partial(
    pl.kernel,
    out_type=jax.ShapeDtypeStruct((NW, N), jnp.float32),
    mesh=_sc_mesh(),
    compiler_params=_sc_params(2),
    scratch_types=[
        pltpu.VMEM((79, K), jnp.int32),
        pltpu.VMEM((N,), jnp.float32),
    ],
)
def _deg_kernel(ei_hbm, zeros_hbm, out_hbm, dst_v, deg_v):
    c = lax.axis_index("c")
    s = lax.axis_index("s")
    wid = s * NC + c
    count, dma_start, off = _deg_split(wid)
    pltpu.sync_copy(zeros_hbm, deg_v)
    pltpu.sync_copy(ei_hbm.at[1].at[pl.ds(dma_start, 79)], dst_v)
    ones = jnp.full((16,), 1.0, dtype=jnp.float32)

    def body(j, carry):
        row = dst_v.at[off + j]
        for g in range(K // 16):
            idx = row[pl.ds(g * 16, 16)]
            plsc.addupdate_scatter(deg_v, [idx], ones)
        return carry

    lax.fori_loop(0, count, body, 0)
    pltpu.sync_copy(deg_v, out_hbm.at[wid])


def _make_agg(H):
    @functools.partial(
        pl.kernel,
        out_type=jax.ShapeDtypeStruct((NC, N, H), jnp.float32),
        mesh=_sc_mesh(),
        compiler_params=_SC_PARAMS,
        scratch_types=[
            pltpu.VMEM((SLAB, K), jnp.int32),
            pltpu.VMEM((SLAB, K), jnp.int32),
            pltpu.VMEM((K, H), jnp.float32),
            pltpu.VMEM((K, H), jnp.float32),
            pltpu.VMEM((K, H), jnp.float32),
            pltpu.VMEM((K, H), jnp.float32),
            pltpu.VMEM_SHARED((N, H), jnp.float32),
            pltpu.SemaphoreType.DMA,
            pltpu.SemaphoreType.DMA,
            pltpu.SemaphoreType.DMA,
            pltpu.SemaphoreType.DMA,
        ],
    )
    def agg(hs_hbm, ei_hbm, zeros_hbm, out_hbm, src_v, dst_v,
            rows0, rows1, rows2, rows3, acc, sem0, sem1, sem2, sem3):
        c = lax.axis_index("c")
        s = lax.axis_index("s")
        count, dma_start, off = _agg_split(c, s)
        rows = (rows0, rows1, rows2, rows3)
        sems = (sem0, sem1, sem2, sem3)

        pltpu.sync_copy(zeros_hbm.at[pl.ds(s * NROW, NROW)],
                        acc.at[pl.ds(s * NROW, NROW)])
        pltpu.sync_copy(ei_hbm.at[0].at[pl.ds(dma_start, SLAB)], src_v)
        pltpu.sync_copy(ei_hbm.at[1].at[pl.ds(dma_start, SLAB)], dst_v)
        plsc.subcore_barrier()

        for q in range(4):
            pltpu.async_copy(hs_hbm.at[src_v.at[off + q]], rows[q], sems[q])

        def body(i, carry):
            j = off + 4 * i
            for q in range(4):
                jq = j + q
                pltpu.make_async_copy(hs_hbm.at[src_v.at[jq]], rows[q],
                                      sems[q]).wait()
                pltpu.sync_copy(rows[q], acc.at[dst_v.at[jq]], add=True)
                jn = jnp.minimum(jq + 4, off + count - 1)
                pltpu.async_copy(hs_hbm.at[src_v.at[jn]], rows[q], sems[q])
            return carry

        lax.fori_loop(0, count // 4, body, 0)
        for q in range(4):
            pltpu.make_async_copy(hs_hbm.at[src_v.at[0]], rows[q],
                                  sems[q]).wait()

        plsc.subcore_barrier()
        pltpu.sync_copy(acc.at[pl.ds(s * NROW, NROW)],
                        out_hbm.at[c].at[pl.ds(s * NROW, NROW)])

    return agg


_agg_h1 = _make_agg(H1)
_agg_h2 = _make_agg(H2)


_DEG_DOT = (((0,), (0,)), ((), ()))


def _dinv_col(degp_block):
    ones = jnp.ones((NW, 1), dtype=jnp.float32)
    deg = lax.dot_general(degp_block, ones, _DEG_DOT,
                          preferred_element_type=jnp.float32)
    return lax.rsqrt(deg + 1.0)


def _dense1_body(x_ref, w_ref, degp_ref, hs_ref):
    dinv = _dinv_col(degp_ref[...])
    h = jnp.dot(x_ref[...], w_ref[...], preferred_element_type=jnp.float32)
    hs_ref[...] = h * dinv


def _dense1(x, W1, degp):
    return pl.pallas_call(
        _dense1_body,
        compiler_params=pltpu.CompilerParams(
            allow_input_fusion=[True] * 3),
        out_shape=jax.ShapeDtypeStruct((N, H1), jnp.float32),
    )(x, W1, degp)


def _dense2_body(p_ref, hs_ref, degp_ref, b_ref, w_ref, out_ref):
    a = p_ref[...]
    dinv = _dinv_col(degp_ref[...])
    t = (a[0] + a[1] + hs_ref[...]) * dinv + b_ref[...]
    t = jnp.maximum(t, 0.0)
    hh = jnp.dot(t, w_ref[...], preferred_element_type=jnp.float32)
    out_ref[...] = hh * dinv


def _dense2(P, hs1, degp, b1r, W2):
    return pl.pallas_call(
        _dense2_body,
        compiler_params=pltpu.CompilerParams(
            allow_input_fusion=[True] * 5),
        out_shape=jax.ShapeDtypeStruct((N, H2), jnp.float32),
    )(P, hs1, degp, b1r, W2)


def _dense3_body(q_ref, hs_ref, degp_ref, b_ref, w_ref, bfc_ref, out_ref):
    a = q_ref[...]
    dinv = _dinv_col(degp_ref[...])
    t = (a[0] + a[1] + hs_ref[...]) * dinv + b_ref[...]
    t = jnp.maximum(t, 0.0)
    logits = jnp.dot(t, w_ref[...], preferred_element_type=jnp.float32)
    logits = logits + bfc_ref[...]
    m = jnp.max(logits, axis=1, keepdims=True)
    e = jnp.exp(logits - m)
    out_ref[...] = e / jnp.sum(e, axis=1, keepdims=True)


def _dense3(Q, hs2, degp, b2r, Wfcp, bfcp):
    return pl.pallas_call(
        _dense3_body,
        compiler_params=pltpu.CompilerParams(
            allow_input_fusion=[True] * 6),
        out_shape=jax.ShapeDtypeStruct((N, CP), jnp.float32),
    )(Q, hs2, degp, b2r, Wfcp, bfcp)


def kernel(x, edge_index, W1, b1, W2, b2, Wfc, bfc):
    ei3 = edge_index.reshape(2, NCHUNKS, K)

    zeros_deg = jnp.zeros((N,), jnp.float32)
    degp = _deg_kernel(ei3, zeros_deg)

    hs1 = _dense1(x, W1, degp)

    zeros1 = jnp.zeros((N, H1), jnp.float32)
    P = _agg_h1(hs1, ei3, zeros1)

    hs2 = _dense2(P, hs1, degp, b1.reshape(1, H1), W2)

    zeros2 = jnp.zeros((N, H2), jnp.float32)
    Q = _agg_h2(hs2, ei3, zeros2)

    Wfcp = jnp.concatenate([Wfc, jnp.zeros((H2, CP - C), jnp.float32)], axis=1)
    bfcp = jnp.concatenate([bfc, jnp.full((CP - C,), -1e30, jnp.float32)])
    out = _dense3(Q, hs2, degp, b2.reshape(1, H2), Wfcp, bfcp.reshape(1, CP))
    return out[:, :C]
